# Initial kernel scaffold; baseline (speedup 1.0000x reference)
import jax
import jax.numpy as jnp
from jax.experimental import pallas as pl


def kernel(x, token_table, pos_table):
    raise NotImplementedError("write your pallas kernel here")


# speedup vs baseline: 4.2491x; 4.2491x over previous
import functools

import jax
import jax.numpy as jnp
from jax import lax
from jax.experimental import pallas as pl
from jax.experimental.pallas import tpu as pltpu
from jax.experimental.pallas import tpu_sc as plsc


def _tok_pos_embed(x3, token_table, pos_table, *, B, L, D, NC, NW):
    seq_per_w = B // NW
    half = L // 2
    mesh = plsc.VectorSubcoreMesh(core_axis_name="c", subcore_axis_name="s")

    @functools.partial(
        pl.kernel,
        mesh=mesh,
        out_type=jax.ShapeDtypeStruct((B, L, D), jnp.float32),
        scratch_types=[
            pltpu.VMEM((2, half), jnp.int32),
            pltpu.VMEM((L, D), jnp.float32),
            pltpu.VMEM((L, D), jnp.float32),
            pltpu.SemaphoreType.DMA,
        ],
    )
    def k(x_hbm, tok_hbm, pos_hbm, out_hbm, idx_v, buf_v, pos_v, sem):
        wid = lax.axis_index("s") * NC + lax.axis_index("c")
        pltpu.sync_copy(pos_hbm, pos_v)

        def body(j, carry):
            s = wid * seq_per_w + j
            pltpu.sync_copy(x_hbm.at[s], idx_v)
            cp1 = pltpu.async_copy(
                tok_hbm.at[idx_v.at[0]], buf_v.at[pl.ds(0, half)], sem)
            cp2 = pltpu.async_copy(
                tok_hbm.at[idx_v.at[1]], buf_v.at[pl.ds(half, half)], sem)
            cp1.wait()
            cp2.wait()

            def add_row(r, c):
                for g in range(D // 16):
                    sl = pl.ds(g * 16, 16)
                    buf_v[r, sl] = buf_v[r, sl] + pos_v[r, sl]
                return c

            lax.fori_loop(0, L, add_row, 0)
            pltpu.sync_copy(buf_v, out_hbm.at[s])
            return carry

        lax.fori_loop(0, seq_per_w, body, 0)

    return k(x3, token_table, pos_table)


def kernel(x, token_table, pos_table):
    B, L = x.shape
    V, D = token_table.shape
    info = plsc.get_sparse_core_info()
    NC, NS = info.num_cores, info.num_subcores
    NW = NC * NS
    x3 = x.astype(jnp.int32).reshape(B, 2, L // 2)
    return _tok_pos_embed(
        x3, token_table, pos_table, B=B, L=L, D=D, NC=NC, NW=NW)


# speedup vs baseline: 7.5759x; 1.7829x over previous
import functools

import jax
import jax.numpy as jnp
from jax import lax
from jax.experimental import pallas as pl
from jax.experimental.pallas import tpu as pltpu
from jax.experimental.pallas import tpu_sc as plsc


def _tok_pos_embed(x4, token_table, pos_table, *, B, L, D, NC, NW):
    seq_per_w = B // NW
    half = L // 2
    mesh = plsc.VectorSubcoreMesh(core_axis_name="c", subcore_axis_name="s")

    @functools.partial(
        pl.kernel,
        mesh=mesh,
        out_type=jax.ShapeDtypeStruct((B, L, D), jnp.float32),
        scratch_types=[
            pltpu.VMEM((2 * seq_per_w, half), jnp.int32),
            pltpu.VMEM((L, D), jnp.float32),
            pltpu.VMEM((L, D), jnp.float32),
            pltpu.VMEM((L, D), jnp.float32),
            pltpu.SemaphoreType.DMA,
            pltpu.SemaphoreType.DMA,
            pltpu.SemaphoreType.DMA,
            pltpu.SemaphoreType.DMA,
        ],
    )
    def k(x_hbm, tok_hbm, pos_hbm, out_hbm, idx_v, buf0, buf1, pos_v,
          g0, g1, o0, o1):
        wid = lax.axis_index("s") * NC + lax.axis_index("c")
        bufs = (buf0, buf1)
        gsems = (g0, g1)
        osems = (o0, o1)

        pltpu.sync_copy(x_hbm.at[wid], idx_v)
        pltpu.sync_copy(pos_hbm, pos_v)

        def start_gather(j, b):
            pltpu.async_copy(
                tok_hbm.at[idx_v.at[2 * j]],
                bufs[b].at[pl.ds(0, half)], gsems[b])
            pltpu.async_copy(
                tok_hbm.at[idx_v.at[2 * j + 1]],
                bufs[b].at[pl.ds(half, half)], gsems[b])

        def wait_gather(b):
            for h in range(2):
                pltpu.make_async_copy(
                    tok_hbm.at[idx_v.at[0]],
                    bufs[b].at[pl.ds(h * half, half)], gsems[b]).wait()

        def wait_out(b):
            pltpu.make_async_copy(bufs[b], out_hbm.at[0], osems[b]).wait()

        start_gather(0, 0)

        def outer(i, carry):
            for b in range(2):
                j = 2 * i + b
                nb = 1 - b

                @pl.when(j + 1 < seq_per_w)
                def _():
                    @pl.when(j >= 1)
                    def _():
                        wait_out(nb)
                    start_gather(j + 1, nb)

                wait_gather(b)

                buf = bufs[b]

                def add_rows(r4, c):
                    for dr in range(4):
                        for g in range(D // 16):
                            sl = pl.ds(g * 16, 16)
                            plsc.addupdate(
                                buf.at[4 * r4 + dr, sl],
                                pos_v[4 * r4 + dr, sl])
                    return c

                lax.fori_loop(0, L // 4, add_rows, 0)
                pltpu.async_copy(
                    buf, out_hbm.at[wid * seq_per_w + j], osems[b])
            return carry

        lax.fori_loop(0, seq_per_w // 2, outer, 0)
        wait_out(0)
        wait_out(1)

    return k(x4, token_table, pos_table)


def kernel(x, token_table, pos_table):
    B, L = x.shape
    V, D = token_table.shape
    info = plsc.get_sparse_core_info()
    NC, NS = info.num_cores, info.num_subcores
    NW = NC * NS
    seq_per_w = B // NW
    x4 = x.astype(jnp.int32).reshape(NW, 2 * seq_per_w, L // 2)
    return _tok_pos_embed(
        x4, token_table, pos_table, B=B, L=L, D=D, NC=NC, NW=NW)
